# baseline (speedup 1.0000x reference)
import jax
import jax.numpy as jnp
from jax import lax
from jax.experimental import pallas as pl
from jax.experimental.pallas import tpu as pltpu


def _net_kernel(x_ref, b_ref, a_ref, o_ref):
    x = x_ref[...].astype(jnp.float32)
    n = x.shape[0]
    num_blocks = b_ref.shape[0]
    inv_n = 1.0 / n
    inv_nm1 = 1.0 / (n - 1)

    for blk in range(num_blocks):
        xb = x.astype(jnp.bfloat16)
        h = jnp.maximum(
            jnp.dot(xb, b_ref[blk], preferred_element_type=jnp.float32), 0.0
        )
        out = jnp.dot(
            h.astype(jnp.bfloat16), a_ref[blk], preferred_element_type=jnp.float32
        )

        s_out = jnp.sum(out, axis=0, keepdims=True)
        ss_out = jnp.sum(out * out, axis=0, keepdims=True)
        var_out = jnp.maximum((ss_out - s_out * s_out * inv_n) * inv_nm1, 0.0)
        x = x + out * lax.rsqrt(jnp.max(var_out))

        s_x = jnp.sum(x, axis=0, keepdims=True)
        ss_x = jnp.sum(x * x, axis=0, keepdims=True)
        mu_x = s_x * inv_n
        var_x = jnp.maximum((ss_x - s_x * mu_x) * inv_nm1, 0.0)
        x = (x - mu_x) * lax.rsqrt(var_x + 1e-16)

    o_ref[...] = x.astype(o_ref.dtype)


def kernel(xs, B_stack, A_stack):
    G, N, D = xs.shape
    nb, _, hid = B_stack.shape
    b16 = B_stack.astype(jnp.bfloat16)
    a16 = A_stack.astype(jnp.bfloat16)
    return pl.pallas_call(
        _net_kernel,
        out_shape=jax.ShapeDtypeStruct((G, N, D), xs.dtype),
        grid=(G,),
        in_specs=[
            pl.BlockSpec((None, N, D), lambda g: (g, 0, 0)),
            pl.BlockSpec((nb, D, hid), lambda g: (0, 0, 0)),
            pl.BlockSpec((nb, hid, D), lambda g: (0, 0, 0)),
        ],
        out_specs=pl.BlockSpec((None, N, D), lambda g: (g, 0, 0)),
        compiler_params=pltpu.CompilerParams(
            dimension_semantics=("parallel",)
        ),
    )(xs, b16, a16)


# speedup vs baseline: 1.2085x; 1.2085x over previous
import jax
import jax.numpy as jnp
from jax import lax
from jax.experimental import pallas as pl
from jax.experimental.pallas import tpu as pltpu

_GB = 4


def _net_kernel(x_ref, b_ref, a_ref, o_ref):
    gb, n, d = x_ref.shape
    num_blocks = b_ref.shape[0]
    inv_n = 1.0 / n
    inv_nm1 = 1.0 / (n - 1)

    x = x_ref[...].astype(jnp.float32)

    for blk in range(num_blocks):
        xb = x.reshape(gb * n, d).astype(jnp.bfloat16)
        h = jnp.maximum(
            jnp.dot(xb, b_ref[blk], preferred_element_type=jnp.float32), 0.0
        )
        out = jnp.dot(
            h.astype(jnp.bfloat16), a_ref[blk], preferred_element_type=jnp.float32
        ).reshape(gb, n, d)

        s_out = jnp.sum(out, axis=1, keepdims=True)
        ss_out = jnp.sum(out * out, axis=1, keepdims=True)
        var_out = jnp.maximum((ss_out - s_out * s_out * inv_n) * inv_nm1, 0.0)
        scale = lax.rsqrt(jnp.max(var_out, axis=2, keepdims=True))
        x = x + out * scale

        s_x = jnp.sum(x, axis=1, keepdims=True)
        ss_x = jnp.sum(x * x, axis=1, keepdims=True)
        mu_x = s_x * inv_n
        var_x = jnp.maximum((ss_x - s_x * mu_x) * inv_nm1, 0.0)
        x = (x - mu_x) * lax.rsqrt(var_x + 1e-16)

    o_ref[...] = x.astype(o_ref.dtype)


def kernel(xs, B_stack, A_stack):
    G, N, D = xs.shape
    nb, _, hid = B_stack.shape
    b16 = B_stack.astype(jnp.bfloat16)
    a16 = A_stack.astype(jnp.bfloat16)
    return pl.pallas_call(
        _net_kernel,
        out_shape=jax.ShapeDtypeStruct((G, N, D), xs.dtype),
        grid=(G // _GB,),
        in_specs=[
            pl.BlockSpec((_GB, N, D), lambda g: (g, 0, 0)),
            pl.BlockSpec((nb, D, hid), lambda g: (0, 0, 0)),
            pl.BlockSpec((nb, hid, D), lambda g: (0, 0, 0)),
        ],
        out_specs=pl.BlockSpec((_GB, N, D), lambda g: (g, 0, 0)),
        compiler_params=pltpu.CompilerParams(
            dimension_semantics=("parallel",)
        ),
    )(xs, b16, a16)


# speedup vs baseline: 1.2360x; 1.0228x over previous
import jax
import jax.numpy as jnp
from jax import lax
from jax.experimental import pallas as pl
from jax.experimental.pallas import tpu as pltpu

_GB = 4
_CHAINS = 2


def _run_chain(x, b_ref, a_ref, num_blocks, n, d):
    gb2 = x.shape[0]
    inv_n = 1.0 / n
    inv_nm1 = 1.0 / (n - 1)

    sx = jnp.sum(x, axis=1, keepdims=True)
    ssx = jnp.sum(x * x, axis=1, keepdims=True)

    for blk in range(num_blocks):
        xb = x.reshape(gb2 * n, d).astype(jnp.bfloat16)
        h = jnp.maximum(
            jnp.dot(xb, b_ref[blk], preferred_element_type=jnp.float32), 0.0
        )
        out = jnp.dot(
            h.astype(jnp.bfloat16), a_ref[blk], preferred_element_type=jnp.float32
        ).reshape(gb2, n, d)

        r1 = jnp.sum(out, axis=1, keepdims=True)
        r2 = jnp.sum(out * out, axis=1, keepdims=True)
        r3 = jnp.sum(x * out, axis=1, keepdims=True)

        var_out = jnp.maximum((r2 - r1 * r1 * inv_n) * inv_nm1, 0.0)
        s = lax.rsqrt(jnp.max(var_out, axis=2, keepdims=True))

        sx2 = sx + s * r1
        ssx2 = ssx + (s * s) * r2 + (2.0 * s) * r3
        mu = sx2 * inv_n
        var_x = jnp.maximum((ssx2 - sx2 * mu) * inv_nm1, 0.0)
        dsc = lax.rsqrt(var_x + 1e-16)

        x = (x + s * out - mu) * dsc

        sx = jnp.zeros_like(sx)
        ssx = (ssx2 - sx2 * mu) * (dsc * dsc)

    return x


def _net_kernel(x_ref, b_ref, a_ref, o_ref):
    gb, n, d = x_ref.shape
    num_blocks = b_ref.shape[0]
    gb2 = gb // _CHAINS

    x = x_ref[...].astype(jnp.float32)
    outs = []
    for c in range(_CHAINS):
        xc = x[c * gb2:(c + 1) * gb2]
        outs.append(_run_chain(xc, b_ref, a_ref, num_blocks, n, d))
    o_ref[...] = jnp.concatenate(outs, axis=0).astype(o_ref.dtype)


def kernel(xs, B_stack, A_stack):
    G, N, D = xs.shape
    nb, _, hid = B_stack.shape
    b16 = B_stack.astype(jnp.bfloat16)
    a16 = A_stack.astype(jnp.bfloat16)
    return pl.pallas_call(
        _net_kernel,
        out_shape=jax.ShapeDtypeStruct((G, N, D), xs.dtype),
        grid=(G // _GB,),
        in_specs=[
            pl.BlockSpec((_GB, N, D), lambda g: (g, 0, 0)),
            pl.BlockSpec((nb, D, hid), lambda g: (0, 0, 0)),
            pl.BlockSpec((nb, hid, D), lambda g: (0, 0, 0)),
        ],
        out_specs=pl.BlockSpec((_GB, N, D), lambda g: (g, 0, 0)),
        compiler_params=pltpu.CompilerParams(
            dimension_semantics=("parallel",)
        ),
    )(xs, b16, a16)


# speedup vs baseline: 1.2611x; 1.0203x over previous
import jax
import jax.numpy as jnp
from jax import lax
from jax.experimental import pallas as pl
from jax.experimental.pallas import tpu as pltpu

_GB = 4


def _mm(x, b_ref, a_ref, blk):
    gb2, n, d = x.shape
    xb = x.reshape(gb2 * n, d).astype(jnp.bfloat16)
    h = jnp.maximum(
        jnp.dot(xb, b_ref[blk], preferred_element_type=jnp.float32), 0.0
    )
    return jnp.dot(
        h.astype(jnp.bfloat16), a_ref[blk], preferred_element_type=jnp.float32
    ).reshape(gb2, n, d)


def _upd(x, out, sx, ssx, inv_n, inv_nm1):
    r1 = jnp.sum(out, axis=1, keepdims=True)
    r2 = jnp.sum(out * out, axis=1, keepdims=True)
    r3 = jnp.sum(x * out, axis=1, keepdims=True)

    var_out = jnp.maximum((r2 - r1 * r1 * inv_n) * inv_nm1, 0.0)
    s = lax.rsqrt(jnp.max(var_out, axis=2, keepdims=True))

    sx2 = sx + s * r1
    ssx2 = ssx + (s * s) * r2 + (2.0 * s) * r3
    mu = sx2 * inv_n
    var_x = jnp.maximum((ssx2 - sx2 * mu) * inv_nm1, 0.0)
    dsc = lax.rsqrt(var_x + 1e-16)

    x = (x + s * out - mu) * dsc
    sx = jnp.zeros_like(sx)
    ssx = (ssx2 - sx2 * mu) * (dsc * dsc)
    return x, sx, ssx


def _net_kernel(x_ref, b_ref, a_ref, o_ref):
    gb, n, d = x_ref.shape
    num_blocks = b_ref.shape[0]
    gb2 = gb // 2
    inv_n = 1.0 / n
    inv_nm1 = 1.0 / (n - 1)

    xa = x_ref[:gb2].astype(jnp.float32)
    xb = x_ref[gb2:].astype(jnp.float32)

    sxa = jnp.sum(xa, axis=1, keepdims=True)
    ssxa = jnp.sum(xa * xa, axis=1, keepdims=True)
    sxb = jnp.sum(xb, axis=1, keepdims=True)
    ssxb = jnp.sum(xb * xb, axis=1, keepdims=True)

    outa = _mm(xa, b_ref, a_ref, 0)
    for blk in range(num_blocks):
        outb = _mm(xb, b_ref, a_ref, blk)
        xa, sxa, ssxa = _upd(xa, outa, sxa, ssxa, inv_n, inv_nm1)
        if blk + 1 < num_blocks:
            outa = _mm(xa, b_ref, a_ref, blk + 1)
        xb, sxb, ssxb = _upd(xb, outb, sxb, ssxb, inv_n, inv_nm1)

    o_ref[:gb2] = xa.astype(o_ref.dtype)
    o_ref[gb2:] = xb.astype(o_ref.dtype)


def kernel(xs, B_stack, A_stack):
    G, N, D = xs.shape
    nb, _, hid = B_stack.shape
    b16 = B_stack.astype(jnp.bfloat16)
    a16 = A_stack.astype(jnp.bfloat16)
    return pl.pallas_call(
        _net_kernel,
        out_shape=jax.ShapeDtypeStruct((G, N, D), xs.dtype),
        grid=(G // _GB,),
        in_specs=[
            pl.BlockSpec((_GB, N, D), lambda g: (g, 0, 0)),
            pl.BlockSpec((nb, D, hid), lambda g: (0, 0, 0)),
            pl.BlockSpec((nb, hid, D), lambda g: (0, 0, 0)),
        ],
        out_specs=pl.BlockSpec((_GB, N, D), lambda g: (g, 0, 0)),
        compiler_params=pltpu.CompilerParams(
            dimension_semantics=("parallel",)
        ),
    )(xs, b16, a16)


# speedup vs baseline: 1.2716x; 1.0083x over previous
import jax
import jax.numpy as jnp
from jax import lax
from jax.experimental import pallas as pl
from jax.experimental.pallas import tpu as pltpu

_GB = 8


def _mm(x, b_ref, a_ref, blk):
    gb2, n, d = x.shape
    xb = x.reshape(gb2 * n, d).astype(jnp.bfloat16)
    h = jnp.maximum(
        jnp.dot(xb, b_ref[blk], preferred_element_type=jnp.float32), 0.0
    )
    return jnp.dot(
        h.astype(jnp.bfloat16), a_ref[blk], preferred_element_type=jnp.float32
    ).reshape(gb2, n, d)


def _upd(x, out, sx, ssx, inv_n, inv_nm1):
    r1 = jnp.sum(out, axis=1, keepdims=True)
    r2 = jnp.sum(out * out, axis=1, keepdims=True)
    r3 = jnp.sum(x * out, axis=1, keepdims=True)

    var_out = jnp.maximum((r2 - r1 * r1 * inv_n) * inv_nm1, 0.0)
    s = lax.rsqrt(jnp.max(var_out, axis=2, keepdims=True))

    sx2 = sx + s * r1
    ssx2 = ssx + (s * s) * r2 + (2.0 * s) * r3
    mu = sx2 * inv_n
    var_x = jnp.maximum((ssx2 - sx2 * mu) * inv_nm1, 0.0)
    dsc = lax.rsqrt(var_x + 1e-16)

    x = (x + s * out - mu) * dsc
    sx = jnp.zeros_like(sx)
    ssx = (ssx2 - sx2 * mu) * (dsc * dsc)
    return x, sx, ssx


def _net_kernel(x_ref, b_ref, a_ref, o_ref):
    gb, n, d = x_ref.shape
    num_blocks = b_ref.shape[0]
    gb2 = gb // 2
    inv_n = 1.0 / n
    inv_nm1 = 1.0 / (n - 1)

    xa = x_ref[:gb2].astype(jnp.float32)
    xb = x_ref[gb2:].astype(jnp.float32)

    sxa = jnp.sum(xa, axis=1, keepdims=True)
    ssxa = jnp.sum(xa * xa, axis=1, keepdims=True)
    sxb = jnp.sum(xb, axis=1, keepdims=True)
    ssxb = jnp.sum(xb * xb, axis=1, keepdims=True)

    outa = _mm(xa, b_ref, a_ref, 0)
    for blk in range(num_blocks):
        outb = _mm(xb, b_ref, a_ref, blk)
        xa, sxa, ssxa = _upd(xa, outa, sxa, ssxa, inv_n, inv_nm1)
        if blk + 1 < num_blocks:
            outa = _mm(xa, b_ref, a_ref, blk + 1)
        xb, sxb, ssxb = _upd(xb, outb, sxb, ssxb, inv_n, inv_nm1)

    o_ref[:gb2] = xa.astype(o_ref.dtype)
    o_ref[gb2:] = xb.astype(o_ref.dtype)


def kernel(xs, B_stack, A_stack):
    G, N, D = xs.shape
    nb, _, hid = B_stack.shape
    b16 = B_stack.astype(jnp.bfloat16)
    a16 = A_stack.astype(jnp.bfloat16)
    return pl.pallas_call(
        _net_kernel,
        out_shape=jax.ShapeDtypeStruct((G, N, D), xs.dtype),
        grid=(G // _GB,),
        in_specs=[
            pl.BlockSpec((_GB, N, D), lambda g: (g, 0, 0)),
            pl.BlockSpec((nb, D, hid), lambda g: (0, 0, 0)),
            pl.BlockSpec((nb, hid, D), lambda g: (0, 0, 0)),
        ],
        out_specs=pl.BlockSpec((_GB, N, D), lambda g: (g, 0, 0)),
        compiler_params=pltpu.CompilerParams(
            dimension_semantics=("parallel",)
        ),
    )(xs, b16, a16)
